# baseline (speedup 1.0000x reference)
import functools

import jax
import jax.numpy as jnp
from jax import lax
from jax.experimental import pallas as pl
from jax.experimental.pallas import tpu as pltpu
from jax.experimental.pallas import tpu_sc as plsc

NC = 2
NS = 16
L = 16
NW = NC * NS

B = 16384
PF = 32
TBLK = 128
B_PER_W = B // NW
GROUPS = B_PER_W // L
WAVE = 1
NBUF = 8
WPG = L // WAVE
AHEAD = 7


def _make_kernel():
  mesh = plsc.VectorSubcoreMesh(
      core_axis_name="c", subcore_axis_name="s", num_cores=NC, num_subcores=NS
  )

  @functools.partial(
      pl.kernel,
      out_type=jax.ShapeDtypeStruct((B,), jnp.float32),
      mesh=mesh,
      scratch_types=[
          pltpu.VMEM((B_PER_W,), jnp.int32),
          pltpu.VMEM((B_PER_W,), jnp.int32),
          pltpu.VMEM((NBUF, WAVE, PF, TBLK), jnp.float32),
          pltpu.VMEM((NBUF, WAVE, PF, TBLK), jnp.float32),
          pltpu.VMEM((PF,), jnp.float32),
          pltpu.VMEM((B_PER_W,), jnp.float32),
          [pltpu.SemaphoreType.DMA] * NBUF,
          [pltpu.SemaphoreType.DMA] * NBUF,
      ],
      compiler_params=pltpu.CompilerParams(needs_layout_passes=False),
  )
  def gmf_kernel(users_hbm, movies_hbm, utab_hbm, mtab_hbm, w_hbm, out_hbm,
                 uidx_v, midx_v, ublk_v, mblk_v, w_v, res_v, usems, msems):
    wid = lax.axis_index("s") * NC + lax.axis_index("c")
    base = wid * B_PER_W

    pltpu.sync_copy(users_hbm.at[pl.ds(base, B_PER_W)], uidx_v)
    pltpu.sync_copy(movies_hbm.at[pl.ds(base, B_PER_W)], midx_v)
    pltpu.sync_copy(w_hbm, w_v)

    w0 = w_v[pl.ds(0, L)]
    w1 = w_v[pl.ds(L, L)]
    lane = lax.iota(jnp.int32, L)
    rows_lo = lane
    rows_hi = lane + L
    mask = ~(TBLK - 1)

    def fire_wave(uvec, mvec, j0, buf):
      for d in range(WAVE):
        ub = jnp.bitwise_and(uvec[j0 + d], mask)
        mb = jnp.bitwise_and(mvec[j0 + d], mask)
        pltpu.async_copy(
            utab_hbm.at[:, pl.ds(pl.multiple_of(ub, TBLK), TBLK)],
            ublk_v.at[buf, d], usems[buf])
        pltpu.async_copy(
            mtab_hbm.at[:, pl.ds(pl.multiple_of(mb, TBLK), TBLK)],
            mblk_v.at[buf, d], msems[buf])

    def drain_wave(buf):
      for d in range(WAVE):
        pltpu.make_async_copy(
            utab_hbm.at[:, pl.ds(0, TBLK)], ublk_v.at[buf, d], usems[buf]
        ).wait()
        pltpu.make_async_copy(
            mtab_hbm.at[:, pl.ds(0, TBLK)], mblk_v.at[buf, d], msems[buf]
        ).wait()

    uvec0 = uidx_v[pl.ds(0, L)]
    mvec0 = midx_v[pl.ds(0, L)]
    for w in range(AHEAD):
      fire_wave(uvec0, mvec0, w * WAVE, w % NBUF)

    def group_body(g, _):
      sl = pl.ds(g * L, L)
      uvec = uidx_v[sl]
      mvec = midx_v[sl]
      ucol = jnp.bitwise_and(uvec, TBLK - 1)
      mcol = jnp.bitwise_and(mvec, TBLK - 1)

      acc = jnp.zeros((L,), jnp.float32)
      for w in range(WPG):
        buf = w % NBUF
        wa = w + AHEAD
        fbuf = wa % NBUF
        if wa < WPG:
          fire_wave(uvec, mvec, wa * WAVE, fbuf)
        else:
          @pl.when(g < GROUPS - 1)
          def _():
            nsl = pl.ds((g + 1) * L, L)
            fire_wave(uidx_v[nsl], midx_v[nsl], (wa - WPG) * WAVE, fbuf)

        drain_wave(buf)
        for d in range(WAVE):
          jj = w * WAVE + d
          uc = jnp.full((L,), ucol[jj], jnp.int32)
          mc = jnp.full((L,), mcol[jj], jnp.int32)
          u0 = plsc.load_gather(ublk_v.at[buf, d], [rows_lo, uc])
          u1 = plsc.load_gather(ublk_v.at[buf, d], [rows_hi, uc])
          m0 = plsc.load_gather(mblk_v.at[buf, d], [rows_lo, mc])
          m1 = plsc.load_gather(mblk_v.at[buf, d], [rows_hi, mc])
          t = u0 * m0 * w0 + u1 * m1 * w1
          s = jnp.sum(t)
          acc = jnp.where(lane == jj, s, acc)
      res_v[sl] = acc
      return ()

    lax.fori_loop(0, GROUPS, group_body, ())
    pltpu.sync_copy(res_v, out_hbm.at[pl.ds(base, B_PER_W)])

  return gmf_kernel


_gmf = _make_kernel()


@jax.jit
def kernel(users, movies, user_table, movie_table, W):
  utab_t = user_table.T
  mtab_t = movie_table.T
  out = _gmf(users, movies, utab_t, mtab_t, W.reshape(PF))
  return out.reshape(B, 1)


# speedup vs baseline: 1.0747x; 1.0747x over previous
import functools

import jax
import jax.numpy as jnp
from jax import lax
from jax.experimental import pallas as pl
from jax.experimental.pallas import tpu as pltpu
from jax.experimental.pallas import tpu_sc as plsc

NC = 2
NS = 16
L = 16
NW = NC * NS

B = 16384
PF = 32
TBLK = 128
B_PER_W = B // NW
GROUPS = B_PER_W // L
WAVE = 2
NBUF = 4
WPG = L // WAVE
AHEAD = 3


def _make_kernel():
  mesh = plsc.VectorSubcoreMesh(
      core_axis_name="c", subcore_axis_name="s", num_cores=NC, num_subcores=NS
  )

  @functools.partial(
      pl.kernel,
      out_type=jax.ShapeDtypeStruct((B,), jnp.float32),
      mesh=mesh,
      scratch_types=[
          pltpu.VMEM((B_PER_W,), jnp.int32),
          pltpu.VMEM((B_PER_W,), jnp.int32),
          pltpu.VMEM((NBUF, WAVE, PF, TBLK), jnp.float32),
          pltpu.VMEM((NBUF, WAVE, PF, TBLK), jnp.float32),
          pltpu.VMEM((PF,), jnp.float32),
          pltpu.VMEM((B_PER_W,), jnp.float32),
          [pltpu.SemaphoreType.DMA] * NBUF,
          [pltpu.SemaphoreType.DMA] * NBUF,
      ],
      compiler_params=pltpu.CompilerParams(needs_layout_passes=False),
  )
  def gmf_kernel(users_hbm, movies_hbm, utab_hbm, mtab_hbm, w_hbm, out_hbm,
                 uidx_v, midx_v, ublk_v, mblk_v, w_v, res_v, usems, msems):
    wid = lax.axis_index("s") * NC + lax.axis_index("c")
    base = wid * B_PER_W

    pltpu.sync_copy(users_hbm.at[pl.ds(base, B_PER_W)], uidx_v)
    pltpu.sync_copy(movies_hbm.at[pl.ds(base, B_PER_W)], midx_v)
    pltpu.sync_copy(w_hbm, w_v)

    w0 = w_v[pl.ds(0, L)]
    w1 = w_v[pl.ds(L, L)]
    lane = lax.iota(jnp.int32, L)
    rows_lo = lane
    rows_hi = lane + L
    mask = ~(TBLK - 1)

    def fire_wave(uvec, mvec, j0, buf):
      for d in range(WAVE):
        ub = jnp.bitwise_and(uvec[j0 + d], mask)
        mb = jnp.bitwise_and(mvec[j0 + d], mask)
        pltpu.async_copy(
            utab_hbm.at[:, pl.ds(pl.multiple_of(ub, TBLK), TBLK)],
            ublk_v.at[buf, d], usems[buf])
        pltpu.async_copy(
            mtab_hbm.at[:, pl.ds(pl.multiple_of(mb, TBLK), TBLK)],
            mblk_v.at[buf, d], msems[buf])

    def drain_wave(buf):
      pltpu.make_async_copy(
          utab_hbm.at[:, pl.ds(0, WAVE * TBLK)], ublk_v.at[buf], usems[buf]
      ).wait()
      pltpu.make_async_copy(
          mtab_hbm.at[:, pl.ds(0, WAVE * TBLK)], mblk_v.at[buf], msems[buf]
      ).wait()

    uvec0 = uidx_v[pl.ds(0, L)]
    mvec0 = midx_v[pl.ds(0, L)]
    for w in range(AHEAD):
      fire_wave(uvec0, mvec0, w * WAVE, w % NBUF)

    def group_body(g, _):
      sl = pl.ds(g * L, L)
      uvec = uidx_v[sl]
      mvec = midx_v[sl]
      ucol = jnp.bitwise_and(uvec, TBLK - 1)
      mcol = jnp.bitwise_and(mvec, TBLK - 1)

      acc = jnp.zeros((L,), jnp.float32)
      for w in range(WPG):
        buf = w % NBUF
        wa = w + AHEAD
        fbuf = wa % NBUF
        if wa < WPG:
          fire_wave(uvec, mvec, wa * WAVE, fbuf)
        else:
          @pl.when(g < GROUPS - 1)
          def _():
            nsl = pl.ds((g + 1) * L, L)
            fire_wave(uidx_v[nsl], midx_v[nsl], (wa - WPG) * WAVE, fbuf)

        drain_wave(buf)
        for d in range(WAVE):
          jj = w * WAVE + d
          uc = jnp.full((L,), ucol[jj], jnp.int32)
          mc = jnp.full((L,), mcol[jj], jnp.int32)
          u0 = plsc.load_gather(ublk_v.at[buf, d], [rows_lo, uc])
          u1 = plsc.load_gather(ublk_v.at[buf, d], [rows_hi, uc])
          m0 = plsc.load_gather(mblk_v.at[buf, d], [rows_lo, mc])
          m1 = plsc.load_gather(mblk_v.at[buf, d], [rows_hi, mc])
          t = u0 * m0 * w0 + u1 * m1 * w1
          s = jnp.sum(t)
          acc = jnp.where(lane == jj, s, acc)
      res_v[sl] = acc
      return ()

    lax.fori_loop(0, GROUPS, group_body, ())
    pltpu.sync_copy(res_v, out_hbm.at[pl.ds(base, B_PER_W)])

  return gmf_kernel


_gmf = _make_kernel()


@jax.jit
def kernel(users, movies, user_table, movie_table, W):
  utab_t = user_table.T
  mtab_t = movie_table.T
  out = _gmf(users, movies, utab_t, mtab_t, W.reshape(PF))
  return out.reshape(B, 1)
